# Initial kernel scaffold; baseline (speedup 1.0000x reference)
import functools

import jax
import jax.numpy as jnp
from jax import lax
from jax.experimental import pallas as pl
from jax.experimental.pallas import tpu as pltpu
from jax.experimental.pallas import tpu_sc as plsc

ALPHA_C = 1000.0
L = 16


def _rsqrt16(x):
    i = lax.bitcast_convert_type(x, jnp.int32)
    i = jnp.int32(0x5F3759DF) - lax.shift_right_arithmetic(i, 1)
    r = lax.bitcast_convert_type(i, jnp.float32)
    for _ in range(3):
        r = r * (1.5 - 0.5 * x * r * r)
    return r


def _make_sc_kernel(n_nodes_pad, n_edges, n_graphs, chunk):
    NC, NS = 2, 16
    NW = NC * NS
    per_w = n_edges // NW
    n_chunks = per_w // chunk
    nodes_per_tile = n_nodes_pad // NS
    vecs_per_chunk = chunk // L
    node_vecs = nodes_per_tile // L

    mesh = plsc.VectorSubcoreMesh(core_axis_name="c", subcore_axis_name="s")

    @functools.partial(
        pl.kernel,
        out_type=jax.ShapeDtypeStruct((NC, 64), jnp.float32),
        mesh=mesh,
        scratch_types=[
            pltpu.VMEM_SHARED((n_nodes_pad,), jnp.float32),
            pltpu.VMEM_SHARED((n_nodes_pad,), jnp.float32),
            pltpu.VMEM_SHARED((n_nodes_pad,), jnp.float32),
            pltpu.VMEM_SHARED((n_nodes_pad,), jnp.int32),
            pltpu.VMEM_SHARED((NS, n_graphs, L), jnp.float32),
            pltpu.VMEM((n_nodes_pad // NS,), jnp.float32),
            pltpu.VMEM((n_nodes_pad // NS,), jnp.int32),
            pltpu.VMEM((64,), jnp.int32),
            pltpu.VMEM((chunk,), jnp.int32),
            pltpu.VMEM((chunk,), jnp.int32),
            pltpu.VMEM((chunk,), jnp.float32),
            pltpu.VMEM((chunk,), jnp.float32),
            pltpu.VMEM((chunk,), jnp.float32),
            pltpu.VMEM((chunk,), jnp.float32),
            pltpu.VMEM((chunk,), jnp.float32),
            pltpu.VMEM((chunk,), jnp.float32),
            pltpu.VMEM((chunk,), jnp.float32),
            pltpu.VMEM((chunk,), jnp.float32),
            pltpu.VMEM((chunk,), jnp.int32),
            pltpu.VMEM((n_graphs, L), jnp.float32),
            pltpu.VMEM((NS, n_graphs, L), jnp.float32),
            pltpu.VMEM((64,), jnp.float32),
        ],
    )
    def sc_kernel(px_hbm, py_hbm, pz_hbm, src_hbm, dst_hbm, r0_hbm, w_hbm,
                  ptr_hbm, out_hbm,
                  px_sh, py_sh, pz_sh, g_sh, acc_sh,
                  stage_v, gstage_v, ptr_v,
                  sidx_v, didx_v, r0_v, w_v,
                  sx_v, sy_v, sz_v, dx_v, dy_v, dz_v, gv_v,
                  acc_v, accall_v, out_v):
        cid = lax.axis_index("c")
        sid = lax.axis_index("s")
        wid = cid * NS + sid
        nodes_per_tile = n_nodes_pad // NS

        node_lo = sid * nodes_per_tile
        for src_ref, dst_ref in ((px_hbm, px_sh), (py_hbm, py_sh),
                                 (pz_hbm, pz_sh)):
            pltpu.sync_copy(src_ref.at[pl.ds(node_lo, nodes_per_tile)], stage_v)
            pltpu.sync_copy(stage_v, dst_ref.at[pl.ds(node_lo, nodes_per_tile)])

        pltpu.sync_copy(ptr_hbm, ptr_v)
        bounds = [ptr_v[j] for j in range(1, n_graphs)]
        lane = lax.iota(jnp.int32, L)

        def g_body(k, _):
            n = node_lo + k * L + lane
            cnt = jnp.zeros((L,), jnp.int32)
            for b in bounds:
                cnt = cnt + jnp.where(n >= b, 1, 0).astype(jnp.int32)
            gstage_v[pl.ds(k * L, L)] = cnt
            return 0

        lax.fori_loop(0, nodes_per_tile // L, g_body, 0)
        pltpu.sync_copy(gstage_v, g_sh.at[pl.ds(node_lo, nodes_per_tile)])

        zero16 = jnp.zeros((L,), jnp.float32)

        def z_body(i, _):
            acc_v[i, :] = zero16
            return 0

        lax.fori_loop(0, n_graphs, z_body, 0)

        plsc.subcore_barrier()

        edge_base = wid * per_w

        def chunk_body(i, _):
            lo = pl.multiple_of(edge_base + i * chunk, 8)
            pltpu.sync_copy(src_hbm.at[pl.ds(lo, chunk)], sidx_v)
            pltpu.sync_copy(dst_hbm.at[pl.ds(lo, chunk)], didx_v)
            pltpu.sync_copy(r0_hbm.at[pl.ds(lo, chunk)], r0_v)
            pltpu.sync_copy(w_hbm.at[pl.ds(lo, chunk)], w_v)
            pltpu.sync_copy(px_sh.at[sidx_v], sx_v)
            pltpu.sync_copy(py_sh.at[sidx_v], sy_v)
            pltpu.sync_copy(pz_sh.at[sidx_v], sz_v)
            pltpu.sync_copy(px_sh.at[didx_v], dx_v)
            pltpu.sync_copy(py_sh.at[didx_v], dy_v)
            pltpu.sync_copy(pz_sh.at[didx_v], dz_v)
            pltpu.sync_copy(g_sh.at[sidx_v], gv_v)

            def vec_body(k, _):
                o = k * L
                ddx = sx_v[pl.ds(o, L)] - dx_v[pl.ds(o, L)]
                ddy = sy_v[pl.ds(o, L)] - dy_v[pl.ds(o, L)]
                ddz = sz_v[pl.ds(o, L)] - dz_v[pl.ds(o, L)]
                d2 = ddx * ddx + ddy * ddy + ddz * ddz + 1e-12
                dist = d2 * _rsqrt16(d2)
                diff = dist - r0_v[pl.ds(o, L)]
                e = (ALPHA_C * w_v[pl.ds(o, L)]) * (diff * diff)
                g = gv_v[pl.ds(o, L)]
                plsc.addupdate_scatter(acc_v, [g, lane], e)
                return 0

            lax.fori_loop(0, vecs_per_chunk, vec_body, 0)
            return 0

        lax.fori_loop(0, n_chunks, chunk_body, 0)

        pltpu.sync_copy(acc_v, acc_sh.at[sid])
        plsc.subcore_barrier()

        @pl.when(sid == 0)
        def _():
            pltpu.sync_copy(acc_sh, accall_v)
            for k in range(64 // L):
                out_v[pl.ds(k * L, L)] = zero16

            def red_body(gi, _):
                tot = accall_v[0, gi, :]
                for t in range(1, NS):
                    tot = tot + accall_v[t, gi, :]
                out_v[gi] = jnp.sum(tot)
                return 0

            lax.fori_loop(0, n_graphs, red_body, 0)
            pltpu.sync_copy(out_v, out_hbm.at[cid])

    return sc_kernel


def kernel(positions, edge_attrs, edge_index, ptr):
    n_nodes = positions.shape[0]
    n_edges = edge_index.shape[1]
    n_graphs = ptr.shape[0] - 1

    n_nodes_pad = ((n_nodes + 127) // 128) * 128
    pad = n_nodes_pad - n_nodes
    px = jnp.pad(positions[:, 0], (0, pad))
    py = jnp.pad(positions[:, 1], (0, pad))
    pz = jnp.pad(positions[:, 2], (0, pad))
    src = edge_index[0]
    dst = edge_index[1]
    r0 = edge_attrs[:, 0]
    w = edge_attrs[:, 1]
    ptr64 = jnp.pad(ptr, (0, 64 - ptr.shape[0]))

    sc = _make_sc_kernel(n_nodes_pad, n_edges, n_graphs, chunk=2000)
    out2 = sc(px, py, pz, src, dst, r0, w, ptr64)
    return (out2[0] + out2[1])[:n_graphs]


# speedup vs baseline: 47.9024x; 47.9024x over previous
import functools

import jax
import jax.numpy as jnp
from jax import lax
from jax.experimental import pallas as pl
from jax.experimental.pallas import tpu as pltpu
from jax.experimental.pallas import tpu_sc as plsc

ALPHA_C = 1000.0
L = 16


def _rsqrt16(x):
    i = lax.bitcast_convert_type(x, jnp.int32)
    i = jnp.int32(0x5F3759DF) - lax.shift_right_arithmetic(i, 1)
    r = lax.bitcast_convert_type(i, jnp.float32)
    for _ in range(3):
        r = r * (1.5 - 0.5 * x * r * r)
    return r


def _make_sc_kernel(n_nodes_pad, n_edges, n_graphs, chunk):
    NC, NS = 2, 16
    NW = NC * NS
    per_w = n_edges // NW
    n_chunks = per_w // chunk
    nodes_per_tile = n_nodes_pad // NS
    vecs_per_chunk = chunk // L
    node_vecs = nodes_per_tile // L

    mesh = plsc.VectorSubcoreMesh(core_axis_name="c", subcore_axis_name="s")

    @functools.partial(
        pl.kernel,
        out_type=jax.ShapeDtypeStruct((NC, 64), jnp.float32),
        mesh=mesh,
        compiler_params=pltpu.CompilerParams(needs_layout_passes=False),
        scratch_types=[
            pltpu.VMEM_SHARED((n_nodes_pad,), jnp.float32),
            pltpu.VMEM_SHARED((n_nodes_pad,), jnp.float32),
            pltpu.VMEM_SHARED((n_nodes_pad,), jnp.float32),
            pltpu.VMEM_SHARED((n_nodes_pad,), jnp.int32),
            pltpu.VMEM_SHARED((NS, n_graphs * L), jnp.float32),
            pltpu.VMEM((n_nodes_pad // NS,), jnp.float32),
            pltpu.VMEM((n_nodes_pad // NS,), jnp.int32),
            pltpu.VMEM((64,), jnp.int32),
            pltpu.VMEM((chunk,), jnp.int32),
            pltpu.VMEM((chunk,), jnp.int32),
            pltpu.VMEM((chunk,), jnp.float32),
            pltpu.VMEM((chunk,), jnp.float32),
            pltpu.VMEM((chunk,), jnp.float32),
            pltpu.VMEM((chunk,), jnp.float32),
            pltpu.VMEM((chunk,), jnp.float32),
            pltpu.VMEM((chunk,), jnp.float32),
            pltpu.VMEM((chunk,), jnp.float32),
            pltpu.VMEM((chunk,), jnp.float32),
            pltpu.VMEM((chunk,), jnp.int32),
            pltpu.VMEM((n_graphs * L,), jnp.float32),
            pltpu.VMEM((NS, n_graphs * L), jnp.float32),
            pltpu.VMEM((64,), jnp.float32),
        ],
    )
    def sc_kernel(px_hbm, py_hbm, pz_hbm, src_hbm, dst_hbm, r0_hbm, w_hbm,
                  ptr_hbm, out_hbm,
                  px_sh, py_sh, pz_sh, g_sh, acc_sh,
                  stage_v, gstage_v, ptr_v,
                  sidx_v, didx_v, r0_v, w_v,
                  sx_v, sy_v, sz_v, dx_v, dy_v, dz_v, gv_v,
                  acc_v, accall_v, out_v):
        cid = lax.axis_index("c")
        sid = lax.axis_index("s")
        wid = cid * NS + sid
        nodes_per_tile = n_nodes_pad // NS

        node_lo = sid * nodes_per_tile
        for src_ref, dst_ref in ((px_hbm, px_sh), (py_hbm, py_sh),
                                 (pz_hbm, pz_sh)):
            pltpu.sync_copy(src_ref.at[pl.ds(node_lo, nodes_per_tile)], stage_v)
            pltpu.sync_copy(stage_v, dst_ref.at[pl.ds(node_lo, nodes_per_tile)])

        pltpu.sync_copy(ptr_hbm, ptr_v)
        ptr_vecs = [ptr_v[pl.ds(k * L, L)] for k in range(64 // L)]
        bounds = [ptr_vecs[j // L][j % L] for j in range(1, n_graphs)]
        lane = lax.iota(jnp.int32, L)

        def g_body(k, _):
            n = node_lo + k * L + lane
            cnt = jnp.zeros((L,), jnp.int32)
            for b in bounds:
                cnt = cnt + jnp.where(n >= b, 1, 0).astype(jnp.int32)
            gstage_v[pl.ds(k * L, L)] = cnt
            return 0

        lax.fori_loop(0, nodes_per_tile // L, g_body, 0)
        pltpu.sync_copy(gstage_v, g_sh.at[pl.ds(node_lo, nodes_per_tile)])

        zero16 = jnp.zeros((L,), jnp.float32)

        def z_body(i, _):
            acc_v[pl.ds(i * L, L)] = zero16
            return 0

        lax.fori_loop(0, n_graphs, z_body, 0)

        plsc.subcore_barrier()

        edge_base = wid * per_w

        def chunk_body(i, _):
            lo = pl.multiple_of(edge_base + i * chunk, 8)
            pltpu.sync_copy(src_hbm.at[pl.ds(lo, chunk)], sidx_v)
            pltpu.sync_copy(dst_hbm.at[pl.ds(lo, chunk)], didx_v)
            pltpu.sync_copy(r0_hbm.at[pl.ds(lo, chunk)], r0_v)
            pltpu.sync_copy(w_hbm.at[pl.ds(lo, chunk)], w_v)
            pltpu.sync_copy(px_sh.at[sidx_v], sx_v)
            pltpu.sync_copy(py_sh.at[sidx_v], sy_v)
            pltpu.sync_copy(pz_sh.at[sidx_v], sz_v)
            pltpu.sync_copy(px_sh.at[didx_v], dx_v)
            pltpu.sync_copy(py_sh.at[didx_v], dy_v)
            pltpu.sync_copy(pz_sh.at[didx_v], dz_v)
            pltpu.sync_copy(g_sh.at[sidx_v], gv_v)

            def vec_body(k, _):
                o = k * L
                ddx = sx_v[pl.ds(o, L)] - dx_v[pl.ds(o, L)]
                ddy = sy_v[pl.ds(o, L)] - dy_v[pl.ds(o, L)]
                ddz = sz_v[pl.ds(o, L)] - dz_v[pl.ds(o, L)]
                d2 = ddx * ddx + ddy * ddy + ddz * ddz + 1e-12
                dist = d2 * _rsqrt16(d2)
                diff = dist - r0_v[pl.ds(o, L)]
                e = (ALPHA_C * w_v[pl.ds(o, L)]) * (diff * diff)
                g = gv_v[pl.ds(o, L)]
                plsc.addupdate_scatter(acc_v, [g * L + lane], e)
                return 0

            lax.fori_loop(0, vecs_per_chunk, vec_body, 0)
            return 0

        lax.fori_loop(0, n_chunks, chunk_body, 0)

        pltpu.sync_copy(acc_v, acc_sh.at[sid])
        plsc.subcore_barrier()

        @pl.when(sid == 0)
        def _():
            pltpu.sync_copy(acc_sh, accall_v)
            for k in range(64 // L):
                row = zero16
                for j in range(L):
                    gi = k * L + j
                    if gi >= n_graphs:
                        break
                    tot = accall_v[0, pl.ds(gi * L, L)]
                    for t in range(1, NS):
                        tot = tot + accall_v[t, pl.ds(gi * L, L)]
                    row = jnp.where(lane == j, jnp.sum(tot), row)
                out_v[pl.ds(k * L, L)] = row
            pltpu.sync_copy(out_v, out_hbm.at[cid])

    return sc_kernel


def kernel(positions, edge_attrs, edge_index, ptr):
    n_nodes = positions.shape[0]
    n_edges = edge_index.shape[1]
    n_graphs = ptr.shape[0] - 1

    n_nodes_pad = ((n_nodes + 127) // 128) * 128
    pad = n_nodes_pad - n_nodes
    px = jnp.pad(positions[:, 0], (0, pad))
    py = jnp.pad(positions[:, 1], (0, pad))
    pz = jnp.pad(positions[:, 2], (0, pad))
    src = edge_index[0]
    dst = edge_index[1]
    r0 = edge_attrs[:, 0]
    w = edge_attrs[:, 1]
    ptr64 = jnp.pad(ptr, (0, 64 - ptr.shape[0]))

    sc = _make_sc_kernel(n_nodes_pad, n_edges, n_graphs, chunk=2000)
    out2 = sc(px, py, pz, src, dst, r0, w, ptr64)
    return (out2[0] + out2[1])[:n_graphs]
